# baseline (speedup 1.0000x reference)
import functools

import jax
import jax.numpy as jnp
from jax import lax
from jax.experimental import pallas as pl
from jax.experimental.pallas import tpu as pltpu
from jax.experimental.pallas import tpu_sc as plsc

VOCAB = 1_000_000
EMBED_DIM = 32
BATCH = 4096
SEQ_LEN = 200

_NC = 2
_NS = 16
_NW = _NC * _NS
_NB = BATCH // 128


def _make_sc_gather():
  mesh = plsc.VectorSubcoreMesh(core_axis_name="c", subcore_axis_name="s")

  @functools.partial(
      pl.kernel,
      out_type=jax.ShapeDtypeStruct((SEQ_LEN, 4, _NB, 8, 128), jnp.float32),
      mesh=mesh,
      compiler_params=pltpu.CompilerParams(
          use_tc_tiling_on_sc=False, needs_layout_passes=False),
      scratch_types=[
          pltpu.VMEM((SEQ_LEN, 128), jnp.int32),
          pltpu.VMEM((2, 128, EMBED_DIM), jnp.float32),
          pltpu.VMEM((2, 4, 8, 128), jnp.float32),
          pltpu.SemaphoreType.DMA,
          pltpu.SemaphoreType.DMA,
          pltpu.SemaphoreType.DMA,
          pltpu.SemaphoreType.DMA,
      ],
  )
  def sc_gather(x_hbm, w_hbm, out_hbm, idx_v, gbuf_v, tbuf_v, g0, g1, s0, s1):
    wid = lax.axis_index("s") * _NC + lax.axis_index("c")
    pltpu.sync_copy(x_hbm.at[:, pl.ds(wid * 128, 128)], idx_v)

    gsems = (g0, g1)
    ssems = (s0, s1)
    iota = lax.iota(jnp.int32, 16)
    row_idx = [iota + bb * 16 for bb in range(8)]

    pltpu.async_copy(w_hbm.at[idx_v.at[0]], gbuf_v.at[0], g0)
    pltpu.async_copy(w_hbm.at[idx_v.at[1]], gbuf_v.at[1], g1)

    @pl.loop(0, SEQ_LEN, step=2)
    def _(j):
      for b in range(2):
        jb = j + b
        pltpu.make_async_copy(
            w_hbm.at[idx_v.at[0]], gbuf_v.at[b], gsems[b]).wait()

        @pl.when(jb >= 2)
        def _():
          pltpu.make_async_copy(
              tbuf_v.at[b], out_hbm.at[0, :, wid], ssems[b]).wait()

        for d in range(EMBED_DIM):
          col_idx = jnp.full((16,), d, jnp.int32)
          for bb in range(8):
            v = plsc.load_gather(gbuf_v.at[b], [row_idx[bb], col_idx])
            tbuf_v[b, d // 8, d % 8, pl.ds(bb * 16, 16)] = v

        pltpu.async_copy(tbuf_v.at[b], out_hbm.at[jb, :, wid], ssems[b])

        @pl.when(jb + 2 < SEQ_LEN)
        def _():
          pltpu.async_copy(
              w_hbm.at[idx_v.at[jb + 2]], gbuf_v.at[b], gsems[b])

    for b in range(2):
      pltpu.make_async_copy(
          tbuf_v.at[b], out_hbm.at[0, :, wid], ssems[b]).wait()

  return sc_gather


_sc_gather = _make_sc_gather()


@jax.jit
def kernel(x, w):
  xt = jnp.swapaxes(x, 0, 1).astype(jnp.int32)
  out5 = _sc_gather(xt, w)
  return out5.transpose(2, 4, 0, 1, 3).reshape(BATCH, SEQ_LEN, EMBED_DIM)


# speedup vs baseline: 1.7417x; 1.7417x over previous
import functools

import jax
import jax.numpy as jnp
from jax import lax
from jax.experimental import pallas as pl
from jax.experimental.pallas import tpu as pltpu
from jax.experimental.pallas import tpu_sc as plsc

VOCAB = 1_000_000
EMBED_DIM = 32
BATCH = 4096
SEQ_LEN = 200

_NC = 2
_NS = 16
_NW = _NC * _NS
_NB = BATCH // 128


def _make_sc_gather():
  mesh = plsc.VectorSubcoreMesh(core_axis_name="c", subcore_axis_name="s")

  @functools.partial(
      pl.kernel,
      out_type=jax.ShapeDtypeStruct((SEQ_LEN, 4, _NB, 8, 128), jnp.float32),
      mesh=mesh,
      compiler_params=pltpu.CompilerParams(
          use_tc_tiling_on_sc=False, needs_layout_passes=False),
      scratch_types=[
          pltpu.VMEM((SEQ_LEN, 128), jnp.int32),
          pltpu.VMEM((2, 128, EMBED_DIM), jnp.float32),
          pltpu.VMEM((2, EMBED_DIM, 129), jnp.float32),
          pltpu.SemaphoreType.DMA,
          pltpu.SemaphoreType.DMA,
          pltpu.SemaphoreType.DMA,
          pltpu.SemaphoreType.DMA,
      ],
  )
  def sc_gather(x_hbm, w_hbm, out_hbm, idx_v, gbuf_v, tbuf_v, g0, g1, s0, s1):
    wid = lax.axis_index("s") * _NC + lax.axis_index("c")
    pltpu.sync_copy(x_hbm.at[:, pl.ds(wid * 128, 128)], idx_v)

    gsems = (g0, g1)
    ssems = (s0, s1)
    iota = lax.iota(jnp.int32, 16)
    d_idx = (iota, iota + 16)

    def store_tile(b, jb):
      for dblk in range(4):
        pltpu.async_copy(
            tbuf_v.at[b, pl.ds(dblk * 8, 8), pl.ds(0, 128)],
            out_hbm.at[jb, dblk, wid], ssems[b])

    def wait_store(b):
      for dblk in range(4):
        pltpu.make_async_copy(
            tbuf_v.at[b, pl.ds(dblk * 8, 8), pl.ds(0, 128)],
            out_hbm.at[0, dblk, wid], ssems[b]).wait()

    pltpu.async_copy(w_hbm.at[idx_v.at[0]], gbuf_v.at[0], g0)
    pltpu.async_copy(w_hbm.at[idx_v.at[1]], gbuf_v.at[1], g1)

    @pl.loop(0, SEQ_LEN, step=2)
    def _(j):
      for b in range(2):
        jb = j + b
        pltpu.make_async_copy(
            w_hbm.at[idx_v.at[0]], gbuf_v.at[b], gsems[b]).wait()

        @pl.when(jb >= 2)
        def _():
          wait_store(b)

        for b0 in range(0, 128, 4):
          vals = [
              (gbuf_v[b, b0 + r, pl.ds(0, 16)], gbuf_v[b, b0 + r, pl.ds(16, 16)])
              for r in range(4)
          ]
          for r in range(4):
            bcol = jnp.full((16,), b0 + r, jnp.int32)
            for h in range(2):
              plsc.store_scatter(
                  tbuf_v.at[b], [d_idx[h], bcol], vals[r][h])

        store_tile(b, jb)

        @pl.when(jb + 2 < SEQ_LEN)
        def _():
          pltpu.async_copy(
              w_hbm.at[idx_v.at[jb + 2]], gbuf_v.at[b], gsems[b])

    for b in range(2):
      wait_store(b)

  return sc_gather


_sc_gather = _make_sc_gather()


@jax.jit
def kernel(x, w):
  xt = jnp.swapaxes(x, 0, 1).astype(jnp.int32)
  out5 = _sc_gather(xt, w)
  return out5.transpose(2, 4, 0, 1, 3).reshape(BATCH, SEQ_LEN, EMBED_DIM)
